# Initial kernel scaffold; baseline (speedup 1.0000x reference)
import jax
import jax.numpy as jnp
from jax.experimental import pallas as pl


def kernel(input, edge_index, rel_type, n_nodes, W, alpha_table, bias):
    raise NotImplementedError("write your pallas kernel here")


# speedup vs baseline: 10.9697x; 10.9697x over previous
import functools

import jax
import jax.numpy as jnp
from jax import lax
from jax.experimental import pallas as pl
from jax.experimental.pallas import tpu as pltpu
from jax.experimental.pallas import tpu_sc as plsc

NC = 2
NS = 16
L = 16
CHUNK = 128


def _sc_aggregate(x, src, dst, rel, alpha_flat):
    n, d = x.shape
    e = src.shape[0]
    assert e % CHUNK == 0
    num_chunks = e // CHUNK
    nw = NC * NS
    full_rounds = num_chunks // nw
    extra = num_chunks - full_rounds * nw
    rows_per_sub = -(-n // (NS * 8)) * 8
    n_pad = rows_per_sub * NS

    mesh = plsc.VectorSubcoreMesh(
        core_axis_name="c", subcore_axis_name="s", num_cores=NC, num_subcores=NS
    )

    @functools.partial(
        pl.kernel,
        out_type=jax.ShapeDtypeStruct((NC, n_pad, d), jnp.float32),
        mesh=mesh,
        scratch_types=[
            pltpu.VMEM((CHUNK,), jnp.int32),
            pltpu.VMEM((CHUNK,), jnp.int32),
            pltpu.VMEM((CHUNK,), jnp.int32),
            pltpu.VMEM((CHUNK,), jnp.float32),
            pltpu.VMEM((CHUNK, d), jnp.float32),
            pltpu.VMEM((CHUNK, d), jnp.float32),
            pltpu.VMEM((256,), jnp.float32),
            pltpu.VMEM_SHARED((n_pad, d), jnp.float32),
            pltpu.SemaphoreType.DMA,
            pltpu.SemaphoreType.DMA,
        ],
        compiler_params=pltpu.CompilerParams(needs_layout_passes=False),
    )
    def agg(x_hbm, src_hbm, dst_hbm, rel_hbm, alpha_hbm, out_hbm,
            src_v, dst_v, rel_v, alp_v, rows_d, rows_s, alpha_v, acc,
            sem0, sem1):
        cid_ax = lax.axis_index("c")
        sid = lax.axis_index("s")
        wid = sid * NC + cid_ax

        pltpu.sync_copy(alpha_hbm, alpha_v)

        zero16 = jnp.zeros((L,), jnp.float32)

        def zrow(i, carry):
            for cc in range(d // L):
                rows_d[i, pl.ds(cc * L, L)] = zero16
            return carry

        lax.fori_loop(0, CHUNK, zrow, 0)
        base_n = sid * rows_per_sub
        n_full = rows_per_sub // CHUNK
        tail = rows_per_sub - n_full * CHUNK
        for i in range(n_full):
            pltpu.sync_copy(rows_d, acc.at[pl.ds(base_n + i * CHUNK, CHUNK)])
        if tail:
            pltpu.sync_copy(rows_d.at[pl.ds(0, tail)],
                            acc.at[pl.ds(base_n + n_full * CHUNK, tail)])
        plsc.subcore_barrier()

        def do_chunk(chunk_id):
            base = chunk_id * CHUNK
            pltpu.sync_copy(src_hbm.at[pl.ds(base, CHUNK)], src_v)
            pltpu.sync_copy(dst_hbm.at[pl.ds(base, CHUNK)], dst_v)
            pltpu.sync_copy(rel_hbm.at[pl.ds(base, CHUNK)], rel_v)
            gd = pltpu.async_copy(x_hbm.at[dst_v], rows_d, sem0)
            gs = pltpu.async_copy(x_hbm.at[src_v], rows_s, sem1)
            for k8 in range(CHUNK // L):
                r16 = rel_v[pl.ds(k8 * L, L)]
                alp_v[pl.ds(k8 * L, L)] = plsc.load_gather(alpha_v, [r16])
            gd.wait()
            gs.wait()

            def edge_group(k, carry):
                a16 = alp_v[pl.ds(k * L, L)]
                for j in range(L):
                    a = jnp.full((L,), a16[j], jnp.float32)
                    row = k * L + j
                    for cc in range(d // L):
                        sl = pl.ds(cc * L, L)
                        rows_d[row, sl] = rows_d[row, sl] * a
                        rows_s[row, sl] = rows_s[row, sl] * a
                return carry

            lax.fori_loop(0, CHUNK // L, edge_group, 0)
            pltpu.sync_copy(rows_d, acc.at[src_v], add=True)
            pltpu.sync_copy(rows_s, acc.at[dst_v], add=True)

        def round_body(t, carry):
            do_chunk(t * nw + wid)
            return carry

        lax.fori_loop(0, full_rounds, round_body, 0)
        if extra:
            @pl.when(wid < extra)
            def _():
                do_chunk(full_rounds * nw + wid)

        plsc.subcore_barrier()
        pltpu.sync_copy(acc.at[pl.ds(base_n, rows_per_sub)],
                        out_hbm.at[cid_ax, pl.ds(base_n, rows_per_sub)])

    return agg(x, src, dst, rel, alpha_flat)


def _tc_combine_matmul(p0, p1, w, bias2d, n):
    d = p0.shape[1]
    blk = 400
    assert n % blk == 0

    def body(p0_ref, p1_ref, w_ref, b_ref, o_ref):
        sup = p0_ref[...] + p1_ref[...]
        o_ref[...] = (
            jnp.dot(sup, w_ref[...], preferred_element_type=jnp.float32)
            + b_ref[...]
        )

    return pl.pallas_call(
        body,
        grid=(n // blk,),
        in_specs=[
            pl.BlockSpec((blk, d), lambda i: (i, 0)),
            pl.BlockSpec((blk, d), lambda i: (i, 0)),
            pl.BlockSpec((d, w.shape[1]), lambda i: (0, 0)),
            pl.BlockSpec((1, w.shape[1]), lambda i: (0, 0)),
        ],
        out_specs=pl.BlockSpec((blk, w.shape[1]), lambda i: (i, 0)),
        out_shape=jax.ShapeDtypeStruct((n, w.shape[1]), jnp.float32),
    )(p0, p1, w, bias2d)


def kernel(input, edge_index, rel_type, n_nodes, W, alpha_table, bias):
    x = input
    src = edge_index[0]
    dst = edge_index[1]
    alpha_flat = jnp.pad(alpha_table[:, 0], (0, 256 - alpha_table.shape[0]))
    partial = _sc_aggregate(x, src, dst, rel_type, alpha_flat)
    return _tc_combine_matmul(partial[0], partial[1], W, bias.reshape(1, -1),
                              x.shape[0])


# speedup vs baseline: 15.1098x; 1.3774x over previous
import functools

import jax
import jax.numpy as jnp
from jax import lax
from jax.experimental import pallas as pl
from jax.experimental.pallas import tpu as pltpu
from jax.experimental.pallas import tpu_sc as plsc

NC = 2
NS = 16
L = 16
CHUNK = 64
NBUF = 3
ATAB = 208


def _sc_aggregate(x, src, dst, rel, alpha_flat):
    n, d = x.shape
    e = src.shape[0]
    assert e % CHUNK == 0
    num_chunks = e // CHUNK
    nw = NC * NS
    full_rounds = num_chunks // nw
    extra = num_chunks - full_rounds * nw
    assert full_rounds % NBUF == 0
    outer = full_rounds // NBUF
    assert n % NS == 0
    zero_per_sub = n // NS
    out_per_sub = -(-n // (NS * 8)) * 8
    n_pad = out_per_sub * NS
    last_rows = n - out_per_sub * (NS - 1)
    assert last_rows > 0 and last_rows % 8 == 0

    mesh = plsc.VectorSubcoreMesh(
        core_axis_name="c", subcore_axis_name="s", num_cores=NC, num_subcores=NS
    )

    @functools.partial(
        pl.kernel,
        out_type=jax.ShapeDtypeStruct((NC, n_pad, d), jnp.float32),
        mesh=mesh,
        scratch_types=[
            [pltpu.VMEM((CHUNK,), jnp.int32) for _ in range(NBUF)],
            [pltpu.VMEM((CHUNK,), jnp.int32) for _ in range(NBUF)],
            [pltpu.VMEM((CHUNK,), jnp.int32) for _ in range(NBUF)],
            pltpu.VMEM((CHUNK,), jnp.float32),
            [pltpu.VMEM((CHUNK, d), jnp.float32) for _ in range(NBUF)],
            [pltpu.VMEM((CHUNK, d), jnp.float32) for _ in range(NBUF)],
            pltpu.VMEM((ATAB,), jnp.float32),
            pltpu.VMEM_SHARED((n, d), jnp.float32),
            [pltpu.SemaphoreType.DMA for _ in range(3)],
            [pltpu.SemaphoreType.DMA for _ in range(NBUF)],
            [pltpu.SemaphoreType.DMA for _ in range(NBUF)],
            [pltpu.SemaphoreType.DMA for _ in range(NBUF)],
            [pltpu.SemaphoreType.DMA for _ in range(NBUF)],
        ],
        compiler_params=pltpu.CompilerParams(needs_layout_passes=False),
    )
    def agg(x_hbm, src_hbm, dst_hbm, rel_hbm, alpha_hbm, out_hbm,
            sv, dv, rl, alp_v, rd, rs, alpha_v, acc,
            isem, gsem_d, gsem_s, ssem_d, ssem_s):
        cid_ax = lax.axis_index("c")
        sid = lax.axis_index("s")
        wid = sid * NC + cid_ax

        pltpu.sync_copy(alpha_hbm, alpha_v)

        zero16 = jnp.zeros((L,), jnp.float32)

        def zrow(i, carry):
            for cc in range(d // L):
                rd[0][i, pl.ds(cc * L, L)] = zero16
            return carry

        lax.fori_loop(0, CHUNK, zrow, 0)
        zbase = sid * zero_per_sub
        n_full = zero_per_sub // CHUNK
        tail = zero_per_sub - n_full * CHUNK
        for i in range(n_full):
            pltpu.sync_copy(rd[0], acc.at[pl.ds(zbase + i * CHUNK, CHUNK)])
        if tail:
            pltpu.sync_copy(rd[0].at[pl.ds(0, tail)],
                            acc.at[pl.ds(zbase + n_full * CHUNK, tail)])
        plsc.subcore_barrier()

        def gather_into(b, cid):
            base = cid * CHUNK
            a1 = pltpu.async_copy(src_hbm.at[pl.ds(base, CHUNK)], sv[b], isem[0])
            a2 = pltpu.async_copy(dst_hbm.at[pl.ds(base, CHUNK)], dv[b], isem[1])
            a3 = pltpu.async_copy(rel_hbm.at[pl.ds(base, CHUNK)], rl[b], isem[2])
            a1.wait()
            a2.wait()
            a3.wait()
            pltpu.async_copy(x_hbm.at[dv[b]], rd[b], gsem_d[b])
            pltpu.async_copy(x_hbm.at[sv[b]], rs[b], gsem_s[b])

        def wait_gathers(b):
            pltpu.make_async_copy(x_hbm.at[dv[b]], rd[b], gsem_d[b]).wait()
            pltpu.make_async_copy(x_hbm.at[sv[b]], rs[b], gsem_s[b]).wait()

        def scale_and_scatter(b):
            for k8 in range(CHUNK // L):
                r16 = rl[b][pl.ds(k8 * L, L)]
                alp_v[pl.ds(k8 * L, L)] = plsc.load_gather(alpha_v, [r16])

            def edge_group(k, carry):
                a16 = alp_v[pl.ds(k * L, L)]
                for j in range(L):
                    a = jnp.full((L,), a16[j], jnp.float32)
                    row = k * L + j
                    for cc in range(d // L):
                        sl = pl.ds(cc * L, L)
                        rd[b][row, sl] = rd[b][row, sl] * a
                        rs[b][row, sl] = rs[b][row, sl] * a
                return carry

            lax.fori_loop(0, CHUNK // L, edge_group, 0)
            pltpu.async_copy(rd[b], acc.at[sv[b]], ssem_d[b], add=True)
            pltpu.async_copy(rs[b], acc.at[dv[b]], ssem_s[b], add=True)

        def wait_scatters(b):
            pltpu.make_async_copy(rd[b], acc.at[sv[b]], ssem_d[b]).wait()
            pltpu.make_async_copy(rs[b], acc.at[dv[b]], ssem_s[b]).wait()

        if extra:
            @pl.when(wid < extra)
            def _():
                gather_into(0, full_rounds * nw + wid)
                wait_gathers(0)
                scale_and_scatter(0)
                wait_scatters(0)

        gather_into(0, 0 * nw + wid)
        gather_into(1, 1 * nw + wid)

        def body(tt, carry):
            for b in range(NBUF):
                t = NBUF * tt + b
                wait_gathers(b)
                pr = (b + 2) % NBUF
                if b == 0:
                    @pl.when(tt >= 1)
                    def _():
                        wait_scatters(pr)
                    gather_into(pr, (t + 2) * nw + wid)
                else:
                    @pl.when(tt < outer - 1)
                    def _():
                        wait_scatters(pr)
                        gather_into(pr, (t + 2) * nw + wid)
                scale_and_scatter(b)
            return carry

        lax.fori_loop(0, outer, body, 0)
        for b in range(NBUF):
            wait_scatters(b)

        plsc.subcore_barrier()
        obase = sid * out_per_sub

        @pl.when(sid < NS - 1)
        def _():
            pltpu.sync_copy(acc.at[pl.ds(obase, out_per_sub)],
                            out_hbm.at[cid_ax, pl.ds(obase, out_per_sub)])

        @pl.when(sid == NS - 1)
        def _():
            lbase = (NS - 1) * out_per_sub
            pltpu.sync_copy(acc.at[pl.ds(lbase, last_rows)],
                            out_hbm.at[cid_ax, pl.ds(lbase, last_rows)])

    return agg(x, src, dst, rel, alpha_flat)


def _tc_combine_matmul(p0, p1, w, bias2d, n):
    d = p0.shape[1]
    blk = 400
    assert n % blk == 0

    def body(p0_ref, p1_ref, w_ref, b_ref, o_ref):
        sup = p0_ref[...] + p1_ref[...]
        o_ref[...] = (
            jnp.dot(sup, w_ref[...], preferred_element_type=jnp.float32)
            + b_ref[...]
        )

    return pl.pallas_call(
        body,
        grid=(n // blk,),
        in_specs=[
            pl.BlockSpec((blk, d), lambda i: (i, 0)),
            pl.BlockSpec((blk, d), lambda i: (i, 0)),
            pl.BlockSpec((d, w.shape[1]), lambda i: (0, 0)),
            pl.BlockSpec((1, w.shape[1]), lambda i: (0, 0)),
        ],
        out_specs=pl.BlockSpec((blk, w.shape[1]), lambda i: (i, 0)),
        out_shape=jax.ShapeDtypeStruct((n, w.shape[1]), jnp.float32),
    )(p0, p1, w, bias2d)


def kernel(input, edge_index, rel_type, n_nodes, W, alpha_table, bias):
    x = input
    alpha_flat = jnp.pad(alpha_table[:, 0], (0, ATAB - alpha_table.shape[0]))
    partial = _sc_aggregate(x, edge_index[0], edge_index[1], rel_type, alpha_flat)
    return _tc_combine_matmul(partial[0], partial[1], W, bias.reshape(1, -1),
                              x.shape[0])
